# baseline (speedup 1.0000x reference)
import dataclasses
import functools

import jax
import jax.numpy as jnp
from jax import lax
from jax.experimental import pallas as pl
from jax.experimental.pallas import tpu as pltpu
from jax.experimental.pallas import tpu_sc as plsc

F32 = jnp.float32

N_CORES = 2
N_SUB = 16
NW = N_CORES * N_SUB
LANES = 16
CHUNK = 112
NRING = 3
BM = 1000


def _cdiv(a, b):
    return -(-a // b)


def _sc_compiler_params():
    cp = pltpu.CompilerParams()
    if "needs_layout_passes" in pltpu.CompilerParams.__dataclass_fields__:
        cp = dataclasses.replace(cp, needs_layout_passes=False)
    return cp


def _pre_body(x_ref, w_ref, a_ref, h_ref, aa_ref, gm_ref):
    h = jnp.dot(x_ref[...], w_ref[...], preferred_element_type=F32)
    h_ref[...] = h
    aa = jnp.dot(h, a_ref[...], preferred_element_type=F32)
    aa_ref[...] = aa
    mb = jnp.broadcast_to(jnp.max(aa, axis=0)[:, None], (8, 128))

    @pl.when(pl.program_id(0) == 0)
    def _():
        gm_ref[...] = mb

    @pl.when(pl.program_id(0) != 0)
    def _():
        gm_ref[...] = jnp.maximum(gm_ref[...], mb)


def _mid_body(x_ref, b_ref, w_ref, a_ref, h_ref, aa_ref, gm_ref):
    x = jax.nn.relu(x_ref[...] + b_ref[...])
    h = jnp.dot(x, w_ref[...], preferred_element_type=F32)
    h_ref[...] = h
    aa = jnp.dot(h, a_ref[...], preferred_element_type=F32)
    aa_ref[...] = aa
    mb = jnp.broadcast_to(jnp.max(aa, axis=0)[:, None], (8, 128))

    @pl.when(pl.program_id(0) == 0)
    def _():
        gm_ref[...] = mb

    @pl.when(pl.program_id(0) != 0)
    def _():
        gm_ref[...] = jnp.maximum(gm_ref[...], mb)


def _post_body(x_ref, b_ref, wc_ref, bc_ref, out_ref, h_ref):
    h = jax.nn.relu(x_ref[...] + b_ref[...])
    h_ref[...] = h
    out_ref[...] = jnp.dot(h, wc_ref[...], preferred_element_type=F32) + bc_ref[...]


def _tc_pre(x, W, A8, n):
    return pl.pallas_call(
        _pre_body,
        grid=(n // BM,),
        in_specs=[
            pl.BlockSpec((BM, 128), lambda i: (i, 0)),
            pl.BlockSpec((128, 128), lambda i: (0, 0)),
            pl.BlockSpec((128, 8), lambda i: (0, 0)),
        ],
        out_specs=[
            pl.BlockSpec((BM, 128), lambda i: (i, 0)),
            pl.BlockSpec((BM, 8), lambda i: (i, 0)),
            pl.BlockSpec((8, 128), lambda i: (0, 0)),
        ],
        out_shape=[
            jax.ShapeDtypeStruct((n, 128), F32),
            jax.ShapeDtypeStruct((n, 8), F32),
            jax.ShapeDtypeStruct((8, 128), F32),
        ],
    )(x, W, A8)


def _tc_mid(xin, b, W, A8, n):
    return pl.pallas_call(
        _mid_body,
        grid=(n // BM,),
        in_specs=[
            pl.BlockSpec((BM, 128), lambda i: (i, 0)),
            pl.BlockSpec((1, 128), lambda i: (0, 0)),
            pl.BlockSpec((128, 128), lambda i: (0, 0)),
            pl.BlockSpec((128, 8), lambda i: (0, 0)),
        ],
        out_specs=[
            pl.BlockSpec((BM, 128), lambda i: (i, 0)),
            pl.BlockSpec((BM, 8), lambda i: (i, 0)),
            pl.BlockSpec((8, 128), lambda i: (0, 0)),
        ],
        out_shape=[
            jax.ShapeDtypeStruct((n, 128), F32),
            jax.ShapeDtypeStruct((n, 8), F32),
            jax.ShapeDtypeStruct((8, 128), F32),
        ],
    )(xin, b, W, A8)


def _tc_post(xin, b, Wc, bc, n, dout):
    return pl.pallas_call(
        _post_body,
        grid=(n // BM,),
        in_specs=[
            pl.BlockSpec((BM, 128), lambda i: (i, 0)),
            pl.BlockSpec((1, 128), lambda i: (0, 0)),
            pl.BlockSpec((128, dout), lambda i: (0, 0)),
            pl.BlockSpec((1, dout), lambda i: (0, 0)),
        ],
        out_specs=[
            pl.BlockSpec((BM, dout), lambda i: (i, 0)),
            pl.BlockSpec((BM, 128), lambda i: (i, 0)),
        ],
        out_shape=[
            jax.ShapeDtypeStruct((n, dout), F32),
            jax.ShapeDtypeStruct((n, 128), F32),
        ],
    )(xin, b, Wc, bc)


def _make_sc_scatter(npad, n_chunks):
    pt_pair = n_chunks // N_SUB
    pt0 = int(round(pt_pair * 0.40 / NRING)) * NRING
    pt1 = pt_pair - pt0
    slab = npad // N_SUB
    assert pt0 % NRING == 0 and pt1 % NRING == 0
    assert slab % 8 == 0 and CHUNK % 8 == 0

    mesh = plsc.VectorSubcoreMesh(core_axis_name="c", subcore_axis_name="s")

    @functools.partial(
        pl.kernel,
        mesh=mesh,
        compiler_params=_sc_compiler_params(),
        out_type=[
            jax.ShapeDtypeStruct((N_CORES, npad, 128), F32),
            jax.ShapeDtypeStruct((N_CORES * npad,), F32),
        ],
        scratch_types=[
            pltpu.VMEM((LANES,), F32),
            pltpu.VMEM((CHUNK,), jnp.int32),
            pltpu.VMEM((CHUNK,), jnp.int32),
            pltpu.VMEM((CHUNK,), jnp.int32),
            pltpu.VMEM((CHUNK,), jnp.int32),
            pltpu.VMEM((CHUNK,), jnp.int32),
            pltpu.VMEM((CHUNK,), jnp.int32),
            pltpu.VMEM((CHUNK,), F32),
            pltpu.VMEM((CHUNK,), F32),
            pltpu.VMEM((CHUNK,), F32),
            pltpu.VMEM((CHUNK,), F32),
            pltpu.VMEM((CHUNK,), F32),
            pltpu.VMEM((CHUNK,), F32),
            pltpu.VMEM((CHUNK, 128), F32),
            pltpu.VMEM((CHUNK, 128), F32),
            pltpu.VMEM((CHUNK, 128), F32),
            pltpu.VMEM_SHARED((npad, 128), F32),
            pltpu.VMEM_SHARED((npad,), F32),
            pltpu.SemaphoreType.DMA,
            pltpu.SemaphoreType.DMA,
            pltpu.SemaphoreType.DMA,
            pltpu.SemaphoreType.DMA,
            pltpu.SemaphoreType.DMA,
            pltpu.SemaphoreType.DMA,
        ],
    )
    def sc_scatter(h_hbm, as_hbm, ad_hbm, g_hbm, src_hbm, dst_hbm,
                   out_hbm, den_hbm,
                   g_t, srcC0, srcC1, srcC2, dstC0, dstC1, dstC2,
                   asg0, asg1, asg2, adg0, adg1, adg2,
                   rows0, rows1, rows2, sh_out, sh_den,
                   gC0, gC1, gC2, sC0, sC1, sC2):
        c = lax.axis_index("c")
        t = lax.axis_index("s")
        srcC = [srcC0, srcC1, srcC2]
        dstC = [dstC0, dstC1, dstC2]
        asg = [asg0, asg1, asg2]
        adg = [adg0, adg1, adg2]
        rows = [rows0, rows1, rows2]
        gC = [gC0, gC1, gC2]
        sC = [sC0, sC1, sC2]

        pltpu.sync_copy(g_hbm, g_t)
        gv = g_t[...]

        z16 = jnp.zeros((LANES,), F32)

        @pl.loop(0, CHUNK)
        def _(r):
            for f in range(128 // LANES):
                rows0[r, pl.ds(f * LANES, LANES)] = z16

        @pl.loop(0, CHUNK // LANES)
        def _(i):
            asg0[pl.ds(i * LANES, LANES)] = z16

        nfull = slab // CHUNK
        rem = slab - nfull * CHUNK
        for q in range(nfull):
            pltpu.sync_copy(rows0, sh_out.at[pl.ds(t * slab + q * CHUNK, CHUNK)])
            pltpu.sync_copy(asg0, sh_den.at[pl.ds(t * slab + q * CHUNK, CHUNK)])
        if rem:
            pltpu.sync_copy(rows0.at[pl.ds(0, rem)],
                            sh_out.at[pl.ds(t * slab + nfull * CHUNK, rem)])
            pltpu.sync_copy(asg0.at[pl.ds(0, rem)],
                            sh_den.at[pl.ds(t * slab + nfull * CHUNK, rem)])

        plsc.subcore_barrier()

        is0 = c == 0
        per_tile = jnp.where(is0, pt0, pt1)
        core_off = jnp.where(is0, 0, N_SUB * pt0)
        cbase = (core_off + t * per_tile) * CHUNK

        def idx_load(k, b):
            sl = pl.ds(cbase + k * CHUNK, CHUNK)
            pltpu.sync_copy(src_hbm.at[sl], srcC[b])
            pltpu.sync_copy(dst_hbm.at[sl], dstC[b])

        def gather_issue(b):
            pltpu.async_copy(h_hbm.at[srcC[b]], rows[b], gC[b])
            pltpu.async_copy(as_hbm.at[srcC[b]], asg[b], gC[b])
            pltpu.async_copy(ad_hbm.at[dstC[b]], adg[b], gC[b])

        def gather_wait(b):
            pltpu.make_async_copy(h_hbm.at[srcC[b]], rows[b], gC[b]).wait()
            pltpu.make_async_copy(as_hbm.at[srcC[b]], asg[b], gC[b]).wait()
            pltpu.make_async_copy(ad_hbm.at[dstC[b]], adg[b], gC[b]).wait()

        def scat_issue(b):
            pltpu.async_copy(rows[b], sh_out.at[dstC[b]], sC[b], add=True)
            pltpu.async_copy(asg[b], sh_den.at[dstC[b]], sC[b], add=True)

        def scat_drain(b):
            pltpu.make_async_copy(rows[b], sh_out.at[dstC[b]], sC[b]).wait()
            pltpu.make_async_copy(asg[b], sh_den.at[dstC[b]], sC[b]).wait()

        for p in range(2):
            idx_load(p, p)
            gather_issue(p)

        @pl.loop(0, per_tile // NRING)
        def _(m):
            for b in range(NRING):
                k = m * NRING + b
                b2 = (b + 2) % NRING
                gather_wait(b)

                @pl.when(k + 2 < per_tile)
                def _():
                    @pl.when(k >= 1)
                    def _():
                        scat_drain(b2)

                    idx_load(k + 2, b2)
                    gather_issue(b2)

                @pl.loop(0, CHUNK // LANES)
                def _(i):
                    sl = pl.ds(i * LANES, LANES)
                    a = asg[b][sl] + adg[b][sl]
                    lr = jnp.maximum(a, 0.2 * a)
                    val = jnp.exp(lr - gv)
                    asg[b][sl] = val
                    for kk in range(LANES):
                        cs = val[kk]
                        r = i * LANES + kk
                        for f in range(128 // LANES):
                            fl = pl.ds(f * LANES, LANES)
                            rows[b][r, fl] = rows[b][r, fl] * cs

                scat_issue(b)

        for b in range(NRING):
            scat_drain(b)

        plsc.subcore_barrier()
        pltpu.sync_copy(sh_out.at[pl.ds(t * slab, slab)],
                        out_hbm.at[c, pl.ds(t * slab, slab)])
        pltpu.sync_copy(sh_den.at[pl.ds(t * slab, slab)],
                        den_hbm.at[pl.ds(c * npad + t * slab, slab)])

    return sc_scatter


def _make_sc_finalize(npad):
    rpw = npad // NW
    assert rpw % LANES == 0

    mesh = plsc.VectorSubcoreMesh(core_axis_name="c", subcore_axis_name="s")

    @functools.partial(
        pl.kernel,
        mesh=mesh,
        compiler_params=_sc_compiler_params(),
        out_type=jax.ShapeDtypeStruct((npad, 128), F32),
        scratch_types=[
            pltpu.VMEM((rpw, 128), F32),
            pltpu.VMEM((rpw, 128), F32),
            pltpu.VMEM((rpw,), F32),
            pltpu.VMEM((rpw,), F32),
        ],
    )
    def sc_finalize(parts_hbm, den_hbm, out_hbm, pa, pb, da, db):
        c = lax.axis_index("c")
        t = lax.axis_index("s")
        w = c * N_SUB + t
        r0 = w * rpw
        pltpu.sync_copy(parts_hbm.at[0, pl.ds(r0, rpw)], pa)
        pltpu.sync_copy(parts_hbm.at[1, pl.ds(r0, rpw)], pb)
        pltpu.sync_copy(den_hbm.at[pl.ds(r0, rpw)], da)
        pltpu.sync_copy(den_hbm.at[pl.ds(npad + r0, rpw)], db)

        @pl.loop(0, rpw // LANES)
        def _(i):
            sl = pl.ds(i * LANES, LANES)
            inv = 1.0 / (da[sl] + db[sl] + 1e-16)
            for kk in range(LANES):
                cs = inv[kk]
                r = i * LANES + kk
                for f in range(128 // LANES):
                    fl = pl.ds(f * LANES, LANES)
                    pa[r, fl] = (pa[r, fl] + pb[r, fl]) * cs

        pltpu.sync_copy(pa, out_hbm.at[pl.ds(r0, rpw)])

    return sc_finalize


def kernel(fts, edge_index, W1, a_src1, a_dst1, b1, W2, a_src2, a_dst2, b2,
           Wc, bc):
    n, d_in = fts.shape
    e = edge_index.shape[1]
    dout = Wc.shape[1]

    npad = _cdiv(n + 1, 8 * NW) * (8 * NW)
    esl = e + n
    n_chunks = _cdiv(_cdiv(esl, CHUNK), NW * NRING) * (NW * NRING)
    epad = n_chunks * CHUNK

    loop = jnp.arange(n, dtype=jnp.int32)
    src = jnp.concatenate([edge_index[0].astype(jnp.int32), loop])
    dst = jnp.concatenate([edge_index[1].astype(jnp.int32), loop])
    pad_dst = n + (jnp.arange(epad - esl, dtype=jnp.int32) % (npad - n))
    srcp = jnp.pad(src, (0, epad - esl))
    dstp = jnp.concatenate([dst, pad_dst])

    def attn_mat(a_s, a_d):
        A8 = jnp.zeros((128, 8), F32)
        return A8.at[:, 0].set(a_s).at[:, 1].set(a_d)

    sc_scatter = _make_sc_scatter(npad, n_chunks)
    sc_finalize = _make_sc_finalize(npad)

    def edge_phase(h, aa, gm):
        asv = jnp.pad(aa[:, 0], (0, npad - n))
        adv = jnp.pad(aa[:, 1], (0, npad - n))
        s = gm[0, 0] + gm[1, 0]
        g16 = jnp.full((LANES,), jnp.maximum(s, 0.2 * s), F32)
        parts, dens = sc_scatter(h, asv, adv, g16, srcp, dstp)
        return sc_finalize(parts, dens)

    h1, aa1, gm1 = _tc_pre(fts, W1, attn_mat(a_src1, a_dst1), n)
    agg1 = edge_phase(h1, aa1, gm1)

    h2, aa2, gm2 = _tc_mid(agg1[:n], b1.reshape(1, 128), W2,
                           attn_mat(a_src2, a_dst2), n)
    agg2 = edge_phase(h2, aa2, gm2)

    out, hf = _tc_post(agg2[:n], b2.reshape(1, 128), Wc.astype(F32),
                       bc.reshape(1, dout), n, dout)
    return (out, hf)


# speedup vs baseline: 1.1695x; 1.1695x over previous
import dataclasses
import functools

import jax
import jax.numpy as jnp
from jax import lax
from jax.experimental import pallas as pl
from jax.experimental.pallas import tpu as pltpu
from jax.experimental.pallas import tpu_sc as plsc

F32 = jnp.float32

N_CORES = 2
N_SUB = 16
NW = N_CORES * N_SUB
LANES = 16
CHUNK = 112
NRING = 3
BM = 1000


def _cdiv(a, b):
    return -(-a // b)


def _sc_compiler_params():
    cp = pltpu.CompilerParams()
    if "needs_layout_passes" in pltpu.CompilerParams.__dataclass_fields__:
        cp = dataclasses.replace(cp, needs_layout_passes=False)
    return cp


def _pre_body(x_ref, w_ref, a_ref, h_ref, aa_ref, gm_ref):
    h = jnp.dot(x_ref[...], w_ref[...], preferred_element_type=F32)
    h_ref[...] = h
    aa = jnp.dot(h, a_ref[...], preferred_element_type=F32)
    aa_ref[...] = aa
    mb = jnp.broadcast_to(jnp.max(aa, axis=0)[:, None], (8, 128))

    @pl.when(pl.program_id(0) == 0)
    def _():
        gm_ref[...] = mb

    @pl.when(pl.program_id(0) != 0)
    def _():
        gm_ref[...] = jnp.maximum(gm_ref[...], mb)


def _mid_body(x_ref, b_ref, w_ref, a_ref, h_ref, aa_ref, gm_ref):
    x = jax.nn.relu(x_ref[...] + b_ref[...])
    h = jnp.dot(x, w_ref[...], preferred_element_type=F32)
    h_ref[...] = h
    aa = jnp.dot(h, a_ref[...], preferred_element_type=F32)
    aa_ref[...] = aa
    mb = jnp.broadcast_to(jnp.max(aa, axis=0)[:, None], (8, 128))

    @pl.when(pl.program_id(0) == 0)
    def _():
        gm_ref[...] = mb

    @pl.when(pl.program_id(0) != 0)
    def _():
        gm_ref[...] = jnp.maximum(gm_ref[...], mb)


def _post_body(x_ref, b_ref, wc_ref, bc_ref, out_ref, h_ref):
    h = jax.nn.relu(x_ref[...] + b_ref[...])
    h_ref[...] = h
    out_ref[...] = jnp.dot(h, wc_ref[...], preferred_element_type=F32) + bc_ref[...]


def _tc_pre(x, W, A8, n):
    return pl.pallas_call(
        _pre_body,
        grid=(n // BM,),
        in_specs=[
            pl.BlockSpec((BM, 128), lambda i: (i, 0)),
            pl.BlockSpec((128, 128), lambda i: (0, 0)),
            pl.BlockSpec((128, 8), lambda i: (0, 0)),
        ],
        out_specs=[
            pl.BlockSpec((BM, 128), lambda i: (i, 0)),
            pl.BlockSpec((BM, 8), lambda i: (i, 0)),
            pl.BlockSpec((8, 128), lambda i: (0, 0)),
        ],
        out_shape=[
            jax.ShapeDtypeStruct((n, 128), F32),
            jax.ShapeDtypeStruct((n, 8), F32),
            jax.ShapeDtypeStruct((8, 128), F32),
        ],
    )(x, W, A8)


def _tc_mid(xin, b, W, A8, n):
    return pl.pallas_call(
        _mid_body,
        grid=(n // BM,),
        in_specs=[
            pl.BlockSpec((BM, 128), lambda i: (i, 0)),
            pl.BlockSpec((1, 128), lambda i: (0, 0)),
            pl.BlockSpec((128, 128), lambda i: (0, 0)),
            pl.BlockSpec((128, 8), lambda i: (0, 0)),
        ],
        out_specs=[
            pl.BlockSpec((BM, 128), lambda i: (i, 0)),
            pl.BlockSpec((BM, 8), lambda i: (i, 0)),
            pl.BlockSpec((8, 128), lambda i: (0, 0)),
        ],
        out_shape=[
            jax.ShapeDtypeStruct((n, 128), F32),
            jax.ShapeDtypeStruct((n, 8), F32),
            jax.ShapeDtypeStruct((8, 128), F32),
        ],
    )(xin, b, W, A8)


def _tc_post(xin, b, Wc, bc, n, dout):
    return pl.pallas_call(
        _post_body,
        grid=(n // BM,),
        in_specs=[
            pl.BlockSpec((BM, 128), lambda i: (i, 0)),
            pl.BlockSpec((1, 128), lambda i: (0, 0)),
            pl.BlockSpec((128, dout), lambda i: (0, 0)),
            pl.BlockSpec((1, dout), lambda i: (0, 0)),
        ],
        out_specs=[
            pl.BlockSpec((BM, dout), lambda i: (i, 0)),
            pl.BlockSpec((BM, 128), lambda i: (i, 0)),
        ],
        out_shape=[
            jax.ShapeDtypeStruct((n, dout), F32),
            jax.ShapeDtypeStruct((n, 128), F32),
        ],
    )(xin, b, Wc, bc)


def _make_sc_scatter(npad, n_chunks):
    pt_pair = n_chunks // N_SUB
    pt0 = int(round(pt_pair * 0.60 / NRING)) * NRING
    pt1 = pt_pair - pt0
    slab = npad // N_SUB
    assert pt0 % NRING == 0 and pt1 % NRING == 0
    assert slab % 8 == 0 and CHUNK % 8 == 0

    mesh = plsc.VectorSubcoreMesh(core_axis_name="c", subcore_axis_name="s")

    @functools.partial(
        pl.kernel,
        mesh=mesh,
        compiler_params=_sc_compiler_params(),
        out_type=[
            jax.ShapeDtypeStruct((N_CORES, npad, 128), F32),
            jax.ShapeDtypeStruct((N_CORES * npad,), F32),
        ],
        scratch_types=[
            pltpu.VMEM((LANES,), F32),
            pltpu.VMEM((CHUNK,), jnp.int32),
            pltpu.VMEM((CHUNK,), jnp.int32),
            pltpu.VMEM((CHUNK,), jnp.int32),
            pltpu.VMEM((CHUNK,), jnp.int32),
            pltpu.VMEM((CHUNK,), jnp.int32),
            pltpu.VMEM((CHUNK,), jnp.int32),
            pltpu.VMEM((CHUNK,), F32),
            pltpu.VMEM((CHUNK,), F32),
            pltpu.VMEM((CHUNK,), F32),
            pltpu.VMEM((CHUNK,), F32),
            pltpu.VMEM((CHUNK,), F32),
            pltpu.VMEM((CHUNK,), F32),
            pltpu.VMEM((CHUNK, 128), F32),
            pltpu.VMEM((CHUNK, 128), F32),
            pltpu.VMEM((CHUNK, 128), F32),
            pltpu.VMEM_SHARED((npad, 128), F32),
            pltpu.VMEM_SHARED((npad,), F32),
            pltpu.SemaphoreType.DMA,
            pltpu.SemaphoreType.DMA,
            pltpu.SemaphoreType.DMA,
            pltpu.SemaphoreType.DMA,
            pltpu.SemaphoreType.DMA,
            pltpu.SemaphoreType.DMA,
        ],
    )
    def sc_scatter(h_hbm, as_hbm, ad_hbm, g_hbm, src_hbm, dst_hbm,
                   out_hbm, den_hbm,
                   g_t, srcC0, srcC1, srcC2, dstC0, dstC1, dstC2,
                   asg0, asg1, asg2, adg0, adg1, adg2,
                   rows0, rows1, rows2, sh_out, sh_den,
                   gC0, gC1, gC2, sC0, sC1, sC2):
        c = lax.axis_index("c")
        t = lax.axis_index("s")
        srcC = [srcC0, srcC1, srcC2]
        dstC = [dstC0, dstC1, dstC2]
        asg = [asg0, asg1, asg2]
        adg = [adg0, adg1, adg2]
        rows = [rows0, rows1, rows2]
        gC = [gC0, gC1, gC2]
        sC = [sC0, sC1, sC2]

        pltpu.sync_copy(g_hbm, g_t)
        gv = g_t[...]

        z16 = jnp.zeros((LANES,), F32)

        @pl.loop(0, CHUNK)
        def _(r):
            for f in range(128 // LANES):
                rows0[r, pl.ds(f * LANES, LANES)] = z16

        @pl.loop(0, CHUNK // LANES)
        def _(i):
            asg0[pl.ds(i * LANES, LANES)] = z16

        nfull = slab // CHUNK
        rem = slab - nfull * CHUNK
        for q in range(nfull):
            pltpu.sync_copy(rows0, sh_out.at[pl.ds(t * slab + q * CHUNK, CHUNK)])
            pltpu.sync_copy(asg0, sh_den.at[pl.ds(t * slab + q * CHUNK, CHUNK)])
        if rem:
            pltpu.sync_copy(rows0.at[pl.ds(0, rem)],
                            sh_out.at[pl.ds(t * slab + nfull * CHUNK, rem)])
            pltpu.sync_copy(asg0.at[pl.ds(0, rem)],
                            sh_den.at[pl.ds(t * slab + nfull * CHUNK, rem)])

        plsc.subcore_barrier()

        is0 = c == 0
        per_tile = jnp.where(is0, pt0, pt1)
        core_off = jnp.where(is0, 0, N_SUB * pt0)
        cbase = (core_off + t * per_tile) * CHUNK

        def idx_load(k, b):
            sl = pl.ds(cbase + k * CHUNK, CHUNK)
            pltpu.sync_copy(src_hbm.at[sl], srcC[b])
            pltpu.sync_copy(dst_hbm.at[sl], dstC[b])

        def gather_issue(b):
            pltpu.async_copy(h_hbm.at[srcC[b]], rows[b], gC[b])
            pltpu.async_copy(as_hbm.at[srcC[b]], asg[b], gC[b])
            pltpu.async_copy(ad_hbm.at[dstC[b]], adg[b], gC[b])

        def gather_wait(b):
            pltpu.make_async_copy(h_hbm.at[srcC[b]], rows[b], gC[b]).wait()
            pltpu.make_async_copy(as_hbm.at[srcC[b]], asg[b], gC[b]).wait()
            pltpu.make_async_copy(ad_hbm.at[dstC[b]], adg[b], gC[b]).wait()

        def scat_issue(b):
            pltpu.async_copy(rows[b], sh_out.at[dstC[b]], sC[b], add=True)
            pltpu.async_copy(asg[b], sh_den.at[dstC[b]], sC[b], add=True)

        def scat_drain(b):
            pltpu.make_async_copy(rows[b], sh_out.at[dstC[b]], sC[b]).wait()
            pltpu.make_async_copy(asg[b], sh_den.at[dstC[b]], sC[b]).wait()

        for p in range(2):
            idx_load(p, p)
            gather_issue(p)

        @pl.loop(0, per_tile // NRING)
        def _(m):
            for b in range(NRING):
                k = m * NRING + b
                b2 = (b + 2) % NRING
                gather_wait(b)

                @pl.when(k + 2 < per_tile)
                def _():
                    @pl.when(k >= 1)
                    def _():
                        scat_drain(b2)

                    idx_load(k + 2, b2)
                    gather_issue(b2)

                @pl.loop(0, CHUNK // LANES)
                def _(i):
                    sl = pl.ds(i * LANES, LANES)
                    a = asg[b][sl] + adg[b][sl]
                    lr = jnp.maximum(a, 0.2 * a)
                    val = jnp.exp(lr - gv)
                    asg[b][sl] = val
                    for kk in range(LANES):
                        cs = val[kk]
                        r = i * LANES + kk
                        for f in range(128 // LANES):
                            fl = pl.ds(f * LANES, LANES)
                            rows[b][r, fl] = rows[b][r, fl] * cs

                scat_issue(b)

        for b in range(NRING):
            scat_drain(b)

        plsc.subcore_barrier()
        pltpu.sync_copy(sh_out.at[pl.ds(t * slab, slab)],
                        out_hbm.at[c, pl.ds(t * slab, slab)])
        pltpu.sync_copy(sh_den.at[pl.ds(t * slab, slab)],
                        den_hbm.at[pl.ds(c * npad + t * slab, slab)])

    return sc_scatter


def _make_sc_finalize(npad):
    rpw = npad // NW
    assert rpw % LANES == 0

    mesh = plsc.VectorSubcoreMesh(core_axis_name="c", subcore_axis_name="s")

    @functools.partial(
        pl.kernel,
        mesh=mesh,
        compiler_params=_sc_compiler_params(),
        out_type=jax.ShapeDtypeStruct((npad, 128), F32),
        scratch_types=[
            pltpu.VMEM((rpw, 128), F32),
            pltpu.VMEM((rpw, 128), F32),
            pltpu.VMEM((rpw,), F32),
            pltpu.VMEM((rpw,), F32),
        ],
    )
    def sc_finalize(parts_hbm, den_hbm, out_hbm, pa, pb, da, db):
        c = lax.axis_index("c")
        t = lax.axis_index("s")
        w = c * N_SUB + t
        r0 = w * rpw
        pltpu.sync_copy(parts_hbm.at[0, pl.ds(r0, rpw)], pa)
        pltpu.sync_copy(parts_hbm.at[1, pl.ds(r0, rpw)], pb)
        pltpu.sync_copy(den_hbm.at[pl.ds(r0, rpw)], da)
        pltpu.sync_copy(den_hbm.at[pl.ds(npad + r0, rpw)], db)

        @pl.loop(0, rpw // LANES)
        def _(i):
            sl = pl.ds(i * LANES, LANES)
            inv = 1.0 / (da[sl] + db[sl] + 1e-16)
            for kk in range(LANES):
                cs = inv[kk]
                r = i * LANES + kk
                for f in range(128 // LANES):
                    fl = pl.ds(f * LANES, LANES)
                    pa[r, fl] = (pa[r, fl] + pb[r, fl]) * cs

        pltpu.sync_copy(pa, out_hbm.at[pl.ds(r0, rpw)])

    return sc_finalize


def kernel(fts, edge_index, W1, a_src1, a_dst1, b1, W2, a_src2, a_dst2, b2,
           Wc, bc):
    n, d_in = fts.shape
    e = edge_index.shape[1]
    dout = Wc.shape[1]

    npad = _cdiv(n + 1, 8 * NW) * (8 * NW)
    esl = e + n
    n_chunks = _cdiv(_cdiv(esl, CHUNK), NW * NRING) * (NW * NRING)
    epad = n_chunks * CHUNK

    loop = jnp.arange(n, dtype=jnp.int32)
    src = jnp.concatenate([edge_index[0].astype(jnp.int32), loop])
    dst = jnp.concatenate([edge_index[1].astype(jnp.int32), loop])
    pad_dst = n + (jnp.arange(epad - esl, dtype=jnp.int32) % (npad - n))
    srcp = jnp.pad(src, (0, epad - esl))
    dstp = jnp.concatenate([dst, pad_dst])

    def attn_mat(a_s, a_d):
        A8 = jnp.zeros((128, 8), F32)
        return A8.at[:, 0].set(a_s).at[:, 1].set(a_d)

    sc_scatter = _make_sc_scatter(npad, n_chunks)
    sc_finalize = _make_sc_finalize(npad)

    def edge_phase(h, aa, gm):
        asv = jnp.pad(aa[:, 0], (0, npad - n))
        adv = jnp.pad(aa[:, 1], (0, npad - n))
        s = gm[0, 0] + gm[1, 0]
        g16 = jnp.full((LANES,), jnp.maximum(s, 0.2 * s), F32)
        parts, dens = sc_scatter(h, asv, adv, g16, srcp, dstp)
        return sc_finalize(parts, dens)

    h1, aa1, gm1 = _tc_pre(fts, W1, attn_mat(a_src1, a_dst1), n)
    agg1 = edge_phase(h1, aa1, gm1)

    h2, aa2, gm2 = _tc_mid(agg1[:n], b1.reshape(1, 128), W2,
                           attn_mat(a_src2, a_dst2), n)
    agg2 = edge_phase(h2, aa2, gm2)

    out, hf = _tc_post(agg2[:n], b2.reshape(1, 128), Wc.astype(F32),
                       bc.reshape(1, dout), n, dout)
    return (out, hf)


# speedup vs baseline: 1.2013x; 1.0272x over previous
import dataclasses
import functools

import jax
import jax.numpy as jnp
from jax import lax
from jax.experimental import pallas as pl
from jax.experimental.pallas import tpu as pltpu
from jax.experimental.pallas import tpu_sc as plsc

F32 = jnp.float32

N_CORES = 2
N_SUB = 16
NW = N_CORES * N_SUB
LANES = 16
CHUNK = 112
NRING = 3
BM = 1000


def _cdiv(a, b):
    return -(-a // b)


def _sc_compiler_params():
    cp = pltpu.CompilerParams()
    if "needs_layout_passes" in pltpu.CompilerParams.__dataclass_fields__:
        cp = dataclasses.replace(cp, needs_layout_passes=False)
    return cp


def _pre_body(x_ref, w_ref, a_ref, h_ref, aa_ref, gm_ref):
    h = jnp.dot(x_ref[...], w_ref[...], preferred_element_type=F32)
    h_ref[...] = h
    aa = jnp.dot(h, a_ref[...], preferred_element_type=F32)
    aa_ref[...] = aa
    mb = jnp.broadcast_to(jnp.max(aa, axis=0)[:, None], (8, 128))

    @pl.when(pl.program_id(0) == 0)
    def _():
        gm_ref[...] = mb

    @pl.when(pl.program_id(0) != 0)
    def _():
        gm_ref[...] = jnp.maximum(gm_ref[...], mb)


def _mid_body(x_ref, b_ref, w_ref, a_ref, h_ref, aa_ref, gm_ref):
    x = jax.nn.relu(x_ref[...] + b_ref[...])
    h = jnp.dot(x, w_ref[...], preferred_element_type=F32)
    h_ref[...] = h
    aa = jnp.dot(h, a_ref[...], preferred_element_type=F32)
    aa_ref[...] = aa
    mb = jnp.broadcast_to(jnp.max(aa, axis=0)[:, None], (8, 128))

    @pl.when(pl.program_id(0) == 0)
    def _():
        gm_ref[...] = mb

    @pl.when(pl.program_id(0) != 0)
    def _():
        gm_ref[...] = jnp.maximum(gm_ref[...], mb)


def _post_body(x_ref, b_ref, wc_ref, bc_ref, out_ref, h_ref):
    h = jax.nn.relu(x_ref[...] + b_ref[...])
    h_ref[...] = h
    out_ref[...] = jnp.dot(h, wc_ref[...], preferred_element_type=F32) + bc_ref[...]


def _tc_pre(x, W, A8, n):
    return pl.pallas_call(
        _pre_body,
        grid=(n // BM,),
        in_specs=[
            pl.BlockSpec((BM, 128), lambda i: (i, 0)),
            pl.BlockSpec((128, 128), lambda i: (0, 0)),
            pl.BlockSpec((128, 8), lambda i: (0, 0)),
        ],
        out_specs=[
            pl.BlockSpec((BM, 128), lambda i: (i, 0)),
            pl.BlockSpec((BM, 8), lambda i: (i, 0)),
            pl.BlockSpec((8, 128), lambda i: (0, 0)),
        ],
        out_shape=[
            jax.ShapeDtypeStruct((n, 128), F32),
            jax.ShapeDtypeStruct((n, 8), F32),
            jax.ShapeDtypeStruct((8, 128), F32),
        ],
    )(x, W, A8)


def _tc_mid(xin, b, W, A8, n):
    return pl.pallas_call(
        _mid_body,
        grid=(n // BM,),
        in_specs=[
            pl.BlockSpec((BM, 128), lambda i: (i, 0)),
            pl.BlockSpec((1, 128), lambda i: (0, 0)),
            pl.BlockSpec((128, 128), lambda i: (0, 0)),
            pl.BlockSpec((128, 8), lambda i: (0, 0)),
        ],
        out_specs=[
            pl.BlockSpec((BM, 128), lambda i: (i, 0)),
            pl.BlockSpec((BM, 8), lambda i: (i, 0)),
            pl.BlockSpec((8, 128), lambda i: (0, 0)),
        ],
        out_shape=[
            jax.ShapeDtypeStruct((n, 128), F32),
            jax.ShapeDtypeStruct((n, 8), F32),
            jax.ShapeDtypeStruct((8, 128), F32),
        ],
    )(xin, b, W, A8)


def _tc_post(xin, b, Wc, bc, n, dout):
    return pl.pallas_call(
        _post_body,
        grid=(n // BM,),
        in_specs=[
            pl.BlockSpec((BM, 128), lambda i: (i, 0)),
            pl.BlockSpec((1, 128), lambda i: (0, 0)),
            pl.BlockSpec((128, dout), lambda i: (0, 0)),
            pl.BlockSpec((1, dout), lambda i: (0, 0)),
        ],
        out_specs=[
            pl.BlockSpec((BM, dout), lambda i: (i, 0)),
            pl.BlockSpec((BM, 128), lambda i: (i, 0)),
        ],
        out_shape=[
            jax.ShapeDtypeStruct((n, dout), F32),
            jax.ShapeDtypeStruct((n, 128), F32),
        ],
    )(xin, b, Wc, bc)


def _make_sc_scatter(npad, n_chunks):
    pt_pair = n_chunks // N_SUB
    pt0 = int(round(pt_pair * 0.60 / NRING)) * NRING
    pt1 = pt_pair - pt0
    slab = npad // N_SUB
    assert pt0 % NRING == 0 and pt1 % NRING == 0
    assert slab % 8 == 0 and CHUNK % 8 == 0

    mesh = plsc.VectorSubcoreMesh(core_axis_name="c", subcore_axis_name="s")

    @functools.partial(
        pl.kernel,
        mesh=mesh,
        compiler_params=_sc_compiler_params(),
        out_type=[
            jax.ShapeDtypeStruct((N_CORES, npad, 128), F32),
            jax.ShapeDtypeStruct((N_CORES * npad,), F32),
        ],
        scratch_types=[
            pltpu.VMEM((LANES,), F32),
            pltpu.VMEM((CHUNK,), jnp.int32),
            pltpu.VMEM((CHUNK,), jnp.int32),
            pltpu.VMEM((CHUNK,), jnp.int32),
            pltpu.VMEM((CHUNK,), jnp.int32),
            pltpu.VMEM((CHUNK,), jnp.int32),
            pltpu.VMEM((CHUNK,), jnp.int32),
            pltpu.VMEM((CHUNK,), F32),
            pltpu.VMEM((CHUNK,), F32),
            pltpu.VMEM((CHUNK,), F32),
            pltpu.VMEM((CHUNK,), F32),
            pltpu.VMEM((CHUNK,), F32),
            pltpu.VMEM((CHUNK,), F32),
            pltpu.VMEM((CHUNK, 128), F32),
            pltpu.VMEM((CHUNK, 128), F32),
            pltpu.VMEM((CHUNK, 128), F32),
            pltpu.VMEM_SHARED((npad, 128), F32),
            pltpu.VMEM_SHARED((npad,), F32),
            pltpu.SemaphoreType.DMA,
            pltpu.SemaphoreType.DMA,
            pltpu.SemaphoreType.DMA,
            pltpu.SemaphoreType.DMA,
            pltpu.SemaphoreType.DMA,
            pltpu.SemaphoreType.DMA,
        ],
    )
    def sc_scatter(h_hbm, as_hbm, ad_hbm, g_hbm, src_hbm, dst_hbm,
                   out_hbm, den_hbm,
                   g_t, srcC0, srcC1, srcC2, dstC0, dstC1, dstC2,
                   asg0, asg1, asg2, adg0, adg1, adg2,
                   rows0, rows1, rows2, sh_out, sh_den,
                   gC0, gC1, gC2, sC0, sC1, sC2):
        c = lax.axis_index("c")
        t = lax.axis_index("s")
        srcC = [srcC0, srcC1, srcC2]
        dstC = [dstC0, dstC1, dstC2]
        asg = [asg0, asg1, asg2]
        adg = [adg0, adg1, adg2]
        rows = [rows0, rows1, rows2]
        gC = [gC0, gC1, gC2]
        sC = [sC0, sC1, sC2]

        pltpu.sync_copy(g_hbm, g_t)
        gv = g_t[...]

        z16 = jnp.zeros((LANES,), F32)

        @pl.loop(0, CHUNK)
        def _(r):
            for f in range(128 // LANES):
                rows0[r, pl.ds(f * LANES, LANES)] = z16

        @pl.loop(0, CHUNK // LANES)
        def _(i):
            asg0[pl.ds(i * LANES, LANES)] = z16

        nfull = slab // CHUNK
        rem = slab - nfull * CHUNK
        for q in range(nfull):
            pltpu.sync_copy(rows0, sh_out.at[pl.ds(t * slab + q * CHUNK, CHUNK)])
            pltpu.sync_copy(asg0, sh_den.at[pl.ds(t * slab + q * CHUNK, CHUNK)])
        if rem:
            pltpu.sync_copy(rows0.at[pl.ds(0, rem)],
                            sh_out.at[pl.ds(t * slab + nfull * CHUNK, rem)])
            pltpu.sync_copy(asg0.at[pl.ds(0, rem)],
                            sh_den.at[pl.ds(t * slab + nfull * CHUNK, rem)])

        plsc.subcore_barrier()

        is0 = c == 0
        per_tile = jnp.where(is0, pt0, pt1)
        core_off = jnp.where(is0, 0, N_SUB * pt0)
        cbase = (core_off + t * per_tile) * CHUNK

        def idx_load(k, b):
            sl = pl.ds(cbase + k * CHUNK, CHUNK)
            pltpu.sync_copy(src_hbm.at[sl], srcC[b])
            pltpu.sync_copy(dst_hbm.at[sl], dstC[b])

        def gather_issue(b):
            pltpu.async_copy(h_hbm.at[srcC[b]], rows[b], gC[b])
            pltpu.async_copy(as_hbm.at[srcC[b]], asg[b], gC[b])
            pltpu.async_copy(ad_hbm.at[dstC[b]], adg[b], gC[b])

        def gather_wait(b):
            pltpu.make_async_copy(h_hbm.at[srcC[b]], rows[b], gC[b]).wait()
            pltpu.make_async_copy(as_hbm.at[srcC[b]], asg[b], gC[b]).wait()
            pltpu.make_async_copy(ad_hbm.at[dstC[b]], adg[b], gC[b]).wait()

        def scat_issue(b):
            pltpu.async_copy(rows[b], sh_out.at[dstC[b]], sC[b], add=True)
            pltpu.async_copy(asg[b], sh_den.at[dstC[b]], sC[b], add=True)

        def scat_drain(b):
            pltpu.make_async_copy(rows[b], sh_out.at[dstC[b]], sC[b]).wait()
            pltpu.make_async_copy(asg[b], sh_den.at[dstC[b]], sC[b]).wait()

        for p in range(2):
            idx_load(p, p)
            gather_issue(p)

        @pl.loop(0, per_tile // NRING)
        def _(m):
            for b in range(NRING):
                k = m * NRING + b
                b2 = (b + 2) % NRING
                gather_wait(b)

                @pl.when(k + 2 < per_tile)
                def _():
                    @pl.when(k >= 1)
                    def _():
                        scat_drain(b2)

                    idx_load(k + 2, b2)
                    gather_issue(b2)

                @pl.loop(0, CHUNK // LANES)
                def _(i):
                    sl = pl.ds(i * LANES, LANES)
                    a = asg[b][sl] + adg[b][sl]
                    lr = jnp.maximum(a, 0.2 * a)
                    val = jnp.exp(lr - gv)
                    asg[b][sl] = val
                    for kk in range(LANES):
                        cs = val[kk]
                        r = i * LANES + kk
                        for f in range(128 // LANES):
                            fl = pl.ds(f * LANES, LANES)
                            rows[b][r, fl] = rows[b][r, fl] * cs

                scat_issue(b)

        for b in range(NRING):
            scat_drain(b)

        plsc.subcore_barrier()
        pltpu.sync_copy(sh_out.at[pl.ds(t * slab, slab)],
                        out_hbm.at[c, pl.ds(t * slab, slab)])
        pltpu.sync_copy(sh_den.at[pl.ds(t * slab, slab)],
                        den_hbm.at[pl.ds(c * npad + t * slab, slab)])

    return sc_scatter


def _make_sc_finalize(npad):
    rpw = npad // NW
    assert rpw % LANES == 0

    mesh = plsc.VectorSubcoreMesh(core_axis_name="c", subcore_axis_name="s")

    @functools.partial(
        pl.kernel,
        mesh=mesh,
        compiler_params=_sc_compiler_params(),
        out_type=jax.ShapeDtypeStruct((npad, 128), F32),
        scratch_types=[
            pltpu.VMEM((rpw, 128), F32),
            pltpu.VMEM((rpw, 128), F32),
            pltpu.VMEM((rpw,), F32),
            pltpu.VMEM((rpw,), F32),
        ],
    )
    def sc_finalize(parts_hbm, den_hbm, out_hbm, pa, pb, da, db):
        c = lax.axis_index("c")
        t = lax.axis_index("s")
        w = c * N_SUB + t
        r0 = w * rpw
        pltpu.sync_copy(parts_hbm.at[0, pl.ds(r0, rpw)], pa)
        pltpu.sync_copy(parts_hbm.at[1, pl.ds(r0, rpw)], pb)
        pltpu.sync_copy(den_hbm.at[pl.ds(r0, rpw)], da)
        pltpu.sync_copy(den_hbm.at[pl.ds(npad + r0, rpw)], db)

        @pl.loop(0, rpw // LANES)
        def _(i):
            sl = pl.ds(i * LANES, LANES)
            inv = 1.0 / (da[sl] + db[sl] + 1e-16)
            for kk in range(LANES):
                cs = inv[kk]
                r = i * LANES + kk
                for f in range(128 // LANES):
                    fl = pl.ds(f * LANES, LANES)
                    pa[r, fl] = (pa[r, fl] + pb[r, fl]) * cs

        pltpu.sync_copy(pa, out_hbm.at[pl.ds(r0, rpw)])

    return sc_finalize


def kernel(fts, edge_index, W1, a_src1, a_dst1, b1, W2, a_src2, a_dst2, b2,
           Wc, bc):
    n, d_in = fts.shape
    e = edge_index.shape[1]
    dout = Wc.shape[1]

    npad = _cdiv(n + 1, 8 * NW) * (8 * NW)
    esl = e + n
    n_chunks = _cdiv(_cdiv(esl, CHUNK), NW * NRING) * (NW * NRING)
    epad = n_chunks * CHUNK

    loop = jnp.arange(n, dtype=jnp.int32)
    src = jnp.concatenate([edge_index[0].astype(jnp.int32), loop])
    dst = jnp.concatenate([edge_index[1].astype(jnp.int32), loop])
    pad_dst = n + (jnp.arange(epad - esl, dtype=jnp.int32) % (npad - n))
    srcp = jnp.pad(src, (0, epad - esl))
    dstp = jnp.concatenate([dst, pad_dst])

    def attn_mat(a_s, a_d):
        A8 = jnp.zeros((128, 8), F32)
        return A8.at[:, 0].set(a_s).at[:, 1].set(a_d)

    sc_scatter = _make_sc_scatter(npad, n_chunks)
    sc_finalize = _make_sc_finalize(npad)

    def edge_phase(h, aa, gm):
        asv = jnp.pad(aa[:, 0], (0, npad - n))
        adv = jnp.pad(aa[:, 1], (0, npad - n))
        s = gm[0, 0] + gm[1, 0]
        g16 = jnp.full((LANES,), jnp.maximum(s, 0.2 * s), F32)
        parts, dens = sc_scatter(h, asv, adv, g16, srcp, dstp)
        return sc_finalize(parts, dens)

    h1, aa1, gm1 = _tc_pre(fts, W1, attn_mat(a_src1, a_dst1), n)
    agg1 = edge_phase(h1, aa1, gm1)

    h2, aa2, gm2 = _tc_mid(agg1, b1.reshape(1, 128), W2,
                           attn_mat(a_src2, a_dst2), n)
    agg2 = edge_phase(h2, aa2, gm2)

    out, hf = _tc_post(agg2, b2.reshape(1, 128), Wc.astype(F32),
                       bc.reshape(1, dout), n, dout)
    return (out, hf)


# speedup vs baseline: 1.2186x; 1.0144x over previous
import dataclasses
import functools

import jax
import jax.numpy as jnp
from jax import lax
from jax.experimental import pallas as pl
from jax.experimental.pallas import tpu as pltpu
from jax.experimental.pallas import tpu_sc as plsc

F32 = jnp.float32

N_CORES = 2
N_SUB = 16
NW = N_CORES * N_SUB
LANES = 16
CHUNK = 112
NRING = 3
BM = 1000


def _cdiv(a, b):
    return -(-a // b)


def _sc_compiler_params():
    cp = pltpu.CompilerParams()
    if "needs_layout_passes" in pltpu.CompilerParams.__dataclass_fields__:
        cp = dataclasses.replace(cp, needs_layout_passes=False)
    return cp


def _pre_body(x_ref, w_ref, a_ref, h_ref, aa_ref, gm_ref):
    h = jnp.dot(x_ref[...], w_ref[...], preferred_element_type=F32)
    h_ref[...] = h
    aa = jnp.dot(h, a_ref[...], preferred_element_type=F32)
    aa_ref[...] = aa
    mb = jnp.broadcast_to(jnp.max(aa, axis=0)[:, None], (8, 128))

    @pl.when(pl.program_id(0) == 0)
    def _():
        gm_ref[...] = mb

    @pl.when(pl.program_id(0) != 0)
    def _():
        gm_ref[...] = jnp.maximum(gm_ref[...], mb)


def _mid_body(x_ref, b_ref, w_ref, a_ref, h_ref, aa_ref, gm_ref):
    x = jax.nn.relu(x_ref[...] + b_ref[...])
    h = jnp.dot(x, w_ref[...], preferred_element_type=F32)
    h_ref[...] = h
    aa = jnp.dot(h, a_ref[...], preferred_element_type=F32)
    aa_ref[...] = aa
    mb = jnp.broadcast_to(jnp.max(aa, axis=0)[:, None], (8, 128))

    @pl.when(pl.program_id(0) == 0)
    def _():
        gm_ref[...] = mb

    @pl.when(pl.program_id(0) != 0)
    def _():
        gm_ref[...] = jnp.maximum(gm_ref[...], mb)


def _post_body(x_ref, b_ref, wc_ref, bc_ref, out_ref, h_ref):
    h = jax.nn.relu(x_ref[...] + b_ref[...])
    h_ref[...] = h
    out_ref[...] = jnp.dot(h, wc_ref[...], preferred_element_type=F32) + bc_ref[...]


def _tc_pre(x, W, A8, n):
    return pl.pallas_call(
        _pre_body,
        grid=(n // BM,),
        in_specs=[
            pl.BlockSpec((BM, 128), lambda i: (i, 0)),
            pl.BlockSpec((128, 128), lambda i: (0, 0)),
            pl.BlockSpec((128, 8), lambda i: (0, 0)),
        ],
        out_specs=[
            pl.BlockSpec((BM, 128), lambda i: (i, 0)),
            pl.BlockSpec((BM, 8), lambda i: (i, 0)),
            pl.BlockSpec((8, 128), lambda i: (0, 0)),
        ],
        out_shape=[
            jax.ShapeDtypeStruct((n, 128), F32),
            jax.ShapeDtypeStruct((n, 8), F32),
            jax.ShapeDtypeStruct((8, 128), F32),
        ],
    )(x, W, A8)


def _tc_mid(xin, b, W, A8, n):
    return pl.pallas_call(
        _mid_body,
        grid=(n // BM,),
        in_specs=[
            pl.BlockSpec((BM, 128), lambda i: (i, 0)),
            pl.BlockSpec((1, 128), lambda i: (0, 0)),
            pl.BlockSpec((128, 128), lambda i: (0, 0)),
            pl.BlockSpec((128, 8), lambda i: (0, 0)),
        ],
        out_specs=[
            pl.BlockSpec((BM, 128), lambda i: (i, 0)),
            pl.BlockSpec((BM, 8), lambda i: (i, 0)),
            pl.BlockSpec((8, 128), lambda i: (0, 0)),
        ],
        out_shape=[
            jax.ShapeDtypeStruct((n, 128), F32),
            jax.ShapeDtypeStruct((n, 8), F32),
            jax.ShapeDtypeStruct((8, 128), F32),
        ],
    )(xin, b, W, A8)


def _tc_post(xin, b, Wc, bc, n, dout):
    return pl.pallas_call(
        _post_body,
        grid=(n // BM,),
        in_specs=[
            pl.BlockSpec((BM, 128), lambda i: (i, 0)),
            pl.BlockSpec((1, 128), lambda i: (0, 0)),
            pl.BlockSpec((128, dout), lambda i: (0, 0)),
            pl.BlockSpec((1, dout), lambda i: (0, 0)),
        ],
        out_specs=[
            pl.BlockSpec((BM, dout), lambda i: (i, 0)),
            pl.BlockSpec((BM, 128), lambda i: (i, 0)),
        ],
        out_shape=[
            jax.ShapeDtypeStruct((n, dout), F32),
            jax.ShapeDtypeStruct((n, 128), F32),
        ],
    )(xin, b, Wc, bc)


def _make_sc_scatter(npad, n_chunks):
    pt_pair = n_chunks // N_SUB
    pt0 = int(round(pt_pair * 0.613 / NRING)) * NRING
    pt1 = pt_pair - pt0
    slab = npad // N_SUB
    assert pt0 % NRING == 0 and pt1 % NRING == 0
    assert slab % 8 == 0 and CHUNK % 8 == 0

    mesh = plsc.VectorSubcoreMesh(core_axis_name="c", subcore_axis_name="s")

    @functools.partial(
        pl.kernel,
        mesh=mesh,
        compiler_params=_sc_compiler_params(),
        out_type=[
            jax.ShapeDtypeStruct((N_CORES, npad, 128), F32),
            jax.ShapeDtypeStruct((N_CORES * npad,), F32),
        ],
        scratch_types=[
            pltpu.VMEM((LANES,), F32),
            pltpu.VMEM((CHUNK,), jnp.int32),
            pltpu.VMEM((CHUNK,), jnp.int32),
            pltpu.VMEM((CHUNK,), jnp.int32),
            pltpu.VMEM((CHUNK,), jnp.int32),
            pltpu.VMEM((CHUNK,), jnp.int32),
            pltpu.VMEM((CHUNK,), jnp.int32),
            pltpu.VMEM((CHUNK,), F32),
            pltpu.VMEM((CHUNK,), F32),
            pltpu.VMEM((CHUNK,), F32),
            pltpu.VMEM((CHUNK,), F32),
            pltpu.VMEM((CHUNK,), F32),
            pltpu.VMEM((CHUNK,), F32),
            pltpu.VMEM((CHUNK, 128), F32),
            pltpu.VMEM((CHUNK, 128), F32),
            pltpu.VMEM((CHUNK, 128), F32),
            pltpu.VMEM_SHARED((npad, 128), F32),
            pltpu.VMEM_SHARED((npad,), F32),
            pltpu.SemaphoreType.DMA,
            pltpu.SemaphoreType.DMA,
            pltpu.SemaphoreType.DMA,
            pltpu.SemaphoreType.DMA,
            pltpu.SemaphoreType.DMA,
            pltpu.SemaphoreType.DMA,
        ],
    )
    def sc_scatter(h_hbm, as_hbm, ad_hbm, g_hbm, src_hbm, dst_hbm,
                   out_hbm, den_hbm,
                   g_t, srcC0, srcC1, srcC2, dstC0, dstC1, dstC2,
                   asg0, asg1, asg2, adg0, adg1, adg2,
                   rows0, rows1, rows2, sh_out, sh_den,
                   gC0, gC1, gC2, sC0, sC1, sC2):
        c = lax.axis_index("c")
        t = lax.axis_index("s")
        srcC = [srcC0, srcC1, srcC2]
        dstC = [dstC0, dstC1, dstC2]
        asg = [asg0, asg1, asg2]
        adg = [adg0, adg1, adg2]
        rows = [rows0, rows1, rows2]
        gC = [gC0, gC1, gC2]
        sC = [sC0, sC1, sC2]

        pltpu.sync_copy(g_hbm, g_t)
        gv = g_t[...]

        z16 = jnp.zeros((LANES,), F32)

        @pl.loop(0, CHUNK)
        def _(r):
            for f in range(128 // LANES):
                rows0[r, pl.ds(f * LANES, LANES)] = z16

        @pl.loop(0, CHUNK // LANES)
        def _(i):
            asg0[pl.ds(i * LANES, LANES)] = z16

        nfull = slab // CHUNK
        rem = slab - nfull * CHUNK
        for q in range(nfull):
            pltpu.sync_copy(rows0, sh_out.at[pl.ds(t * slab + q * CHUNK, CHUNK)])
            pltpu.sync_copy(asg0, sh_den.at[pl.ds(t * slab + q * CHUNK, CHUNK)])
        if rem:
            pltpu.sync_copy(rows0.at[pl.ds(0, rem)],
                            sh_out.at[pl.ds(t * slab + nfull * CHUNK, rem)])
            pltpu.sync_copy(asg0.at[pl.ds(0, rem)],
                            sh_den.at[pl.ds(t * slab + nfull * CHUNK, rem)])

        plsc.subcore_barrier()

        is0 = c == 0
        per_tile = jnp.where(is0, pt0, pt1)
        core_off = jnp.where(is0, 0, N_SUB * pt0)
        cbase = (core_off + t * per_tile) * CHUNK

        def idx_load(k, b):
            sl = pl.ds(cbase + k * CHUNK, CHUNK)
            pltpu.sync_copy(src_hbm.at[sl], srcC[b])
            pltpu.sync_copy(dst_hbm.at[sl], dstC[b])

        def gather_issue(b):
            pltpu.async_copy(h_hbm.at[srcC[b]], rows[b], gC[b])
            pltpu.async_copy(as_hbm.at[srcC[b]], asg[b], gC[b])
            pltpu.async_copy(ad_hbm.at[dstC[b]], adg[b], gC[b])

        def gather_wait(b):
            pltpu.make_async_copy(h_hbm.at[srcC[b]], rows[b], gC[b]).wait()
            pltpu.make_async_copy(as_hbm.at[srcC[b]], asg[b], gC[b]).wait()
            pltpu.make_async_copy(ad_hbm.at[dstC[b]], adg[b], gC[b]).wait()

        def scat_issue(b):
            pltpu.async_copy(rows[b], sh_out.at[dstC[b]], sC[b], add=True)
            pltpu.async_copy(asg[b], sh_den.at[dstC[b]], sC[b], add=True)

        def scat_drain(b):
            pltpu.make_async_copy(rows[b], sh_out.at[dstC[b]], sC[b]).wait()
            pltpu.make_async_copy(asg[b], sh_den.at[dstC[b]], sC[b]).wait()

        for p in range(2):
            idx_load(p, p)
            gather_issue(p)

        @pl.loop(0, per_tile // NRING)
        def _(m):
            for b in range(NRING):
                k = m * NRING + b
                b2 = (b + 2) % NRING
                gather_wait(b)

                @pl.when(k + 2 < per_tile)
                def _():
                    @pl.when(k >= 1)
                    def _():
                        scat_drain(b2)

                    idx_load(k + 2, b2)
                    gather_issue(b2)

                @pl.loop(0, CHUNK // LANES)
                def _(i):
                    sl = pl.ds(i * LANES, LANES)
                    a = asg[b][sl] + adg[b][sl]
                    lr = jnp.maximum(a, 0.2 * a)
                    val = jnp.exp(lr - gv)
                    asg[b][sl] = val
                    for kk in range(LANES):
                        cs = val[kk]
                        r = i * LANES + kk
                        for f in range(128 // LANES):
                            fl = pl.ds(f * LANES, LANES)
                            rows[b][r, fl] = rows[b][r, fl] * cs

                scat_issue(b)

        for b in range(NRING):
            scat_drain(b)

        plsc.subcore_barrier()
        pltpu.sync_copy(sh_out.at[pl.ds(t * slab, slab)],
                        out_hbm.at[c, pl.ds(t * slab, slab)])
        pltpu.sync_copy(sh_den.at[pl.ds(t * slab, slab)],
                        den_hbm.at[pl.ds(c * npad + t * slab, slab)])

    return sc_scatter


def _make_sc_finalize(npad):
    rpw = npad // NW
    assert rpw % LANES == 0

    mesh = plsc.VectorSubcoreMesh(core_axis_name="c", subcore_axis_name="s")

    @functools.partial(
        pl.kernel,
        mesh=mesh,
        compiler_params=_sc_compiler_params(),
        out_type=jax.ShapeDtypeStruct((npad, 128), F32),
        scratch_types=[
            pltpu.VMEM((rpw, 128), F32),
            pltpu.VMEM((rpw, 128), F32),
            pltpu.VMEM((rpw,), F32),
            pltpu.VMEM((rpw,), F32),
        ],
    )
    def sc_finalize(parts_hbm, den_hbm, out_hbm, pa, pb, da, db):
        c = lax.axis_index("c")
        t = lax.axis_index("s")
        w = c * N_SUB + t
        r0 = w * rpw
        pltpu.sync_copy(parts_hbm.at[0, pl.ds(r0, rpw)], pa)
        pltpu.sync_copy(parts_hbm.at[1, pl.ds(r0, rpw)], pb)
        pltpu.sync_copy(den_hbm.at[pl.ds(r0, rpw)], da)
        pltpu.sync_copy(den_hbm.at[pl.ds(npad + r0, rpw)], db)

        @pl.loop(0, rpw // LANES)
        def _(i):
            sl = pl.ds(i * LANES, LANES)
            inv = 1.0 / (da[sl] + db[sl] + 1e-16)
            for kk in range(LANES):
                cs = inv[kk]
                r = i * LANES + kk
                for f in range(128 // LANES):
                    fl = pl.ds(f * LANES, LANES)
                    pa[r, fl] = (pa[r, fl] + pb[r, fl]) * cs

        pltpu.sync_copy(pa, out_hbm.at[pl.ds(r0, rpw)])

    return sc_finalize


def kernel(fts, edge_index, W1, a_src1, a_dst1, b1, W2, a_src2, a_dst2, b2,
           Wc, bc):
    n, d_in = fts.shape
    e = edge_index.shape[1]
    dout = Wc.shape[1]

    npad = _cdiv(n + 1, 8 * NW) * (8 * NW)
    esl = e + n
    n_chunks = _cdiv(_cdiv(esl, CHUNK), NW * NRING) * (NW * NRING)
    epad = n_chunks * CHUNK

    loop = jnp.arange(n, dtype=jnp.int32)
    src = jnp.concatenate([edge_index[0].astype(jnp.int32), loop])
    dst = jnp.concatenate([edge_index[1].astype(jnp.int32), loop])
    pad_dst = n + (jnp.arange(epad - esl, dtype=jnp.int32) % (npad - n))
    srcp = jnp.pad(src, (0, epad - esl))
    dstp = jnp.concatenate([dst, pad_dst])

    def attn_mat(a_s, a_d):
        A8 = jnp.zeros((128, 8), F32)
        return A8.at[:, 0].set(a_s).at[:, 1].set(a_d)

    sc_scatter = _make_sc_scatter(npad, n_chunks)
    sc_finalize = _make_sc_finalize(npad)

    def edge_phase(h, aa, gm):
        asv = jnp.pad(aa[:, 0], (0, npad - n))
        adv = jnp.pad(aa[:, 1], (0, npad - n))
        s = gm[0, 0] + gm[1, 0]
        g16 = jnp.full((LANES,), jnp.maximum(s, 0.2 * s), F32)
        parts, dens = sc_scatter(h, asv, adv, g16, srcp, dstp)
        return sc_finalize(parts, dens)

    h1, aa1, gm1 = _tc_pre(fts, W1, attn_mat(a_src1, a_dst1), n)
    agg1 = edge_phase(h1, aa1, gm1)

    h2, aa2, gm2 = _tc_mid(agg1, b1.reshape(1, 128), W2,
                           attn_mat(a_src2, a_dst2), n)
    agg2 = edge_phase(h2, aa2, gm2)

    out, hf = _tc_post(agg2, b2.reshape(1, 128), Wc.astype(F32),
                       bc.reshape(1, dout), n, dout)
    return (out, hf)


# speedup vs baseline: 1.2618x; 1.0354x over previous
import dataclasses
import functools

import jax
import jax.numpy as jnp
from jax import lax
from jax.experimental import pallas as pl
from jax.experimental.pallas import tpu as pltpu
from jax.experimental.pallas import tpu_sc as plsc

F32 = jnp.float32

N_CORES = 2
N_SUB = 16
NW = N_CORES * N_SUB
LANES = 16
CHUNK = 112
NRING = 3
BM = 1000


def _cdiv(a, b):
    return -(-a // b)


def _sc_compiler_params():
    cp = pltpu.CompilerParams()
    if "needs_layout_passes" in pltpu.CompilerParams.__dataclass_fields__:
        cp = dataclasses.replace(cp, needs_layout_passes=False)
    return cp


def _pre_body(x_ref, w_ref, a_ref, h_ref, aa_ref, gm_ref):
    h = jnp.dot(x_ref[...], w_ref[...], preferred_element_type=F32)
    h_ref[...] = h
    aa = jnp.dot(h, a_ref[...], preferred_element_type=F32)
    aa_ref[...] = aa
    mb = jnp.broadcast_to(jnp.max(aa, axis=0)[:, None], (8, 128))

    @pl.when(pl.program_id(0) == 0)
    def _():
        gm_ref[...] = mb

    @pl.when(pl.program_id(0) != 0)
    def _():
        gm_ref[...] = jnp.maximum(gm_ref[...], mb)


def _mid_body(x_ref, b_ref, w_ref, a_ref, h_ref, aa_ref, gm_ref):
    x = jax.nn.relu(x_ref[...] + b_ref[...])
    h = jnp.dot(x, w_ref[...], preferred_element_type=F32)
    h_ref[...] = h
    aa = jnp.dot(h, a_ref[...], preferred_element_type=F32)
    aa_ref[...] = aa
    mb = jnp.broadcast_to(jnp.max(aa, axis=0)[:, None], (8, 128))

    @pl.when(pl.program_id(0) == 0)
    def _():
        gm_ref[...] = mb

    @pl.when(pl.program_id(0) != 0)
    def _():
        gm_ref[...] = jnp.maximum(gm_ref[...], mb)


def _post_body(x_ref, b_ref, wc_ref, bc_ref, out_ref, h_ref):
    h = jax.nn.relu(x_ref[...] + b_ref[...])
    h_ref[...] = h
    out_ref[...] = jnp.dot(h, wc_ref[...], preferred_element_type=F32) + bc_ref[...]


def _tc_pre(x, W, A8, n):
    return pl.pallas_call(
        _pre_body,
        grid=(n // BM,),
        in_specs=[
            pl.BlockSpec((BM, 128), lambda i: (i, 0)),
            pl.BlockSpec((128, 128), lambda i: (0, 0)),
            pl.BlockSpec((128, 8), lambda i: (0, 0)),
        ],
        out_specs=[
            pl.BlockSpec((BM, 128), lambda i: (i, 0)),
            pl.BlockSpec((BM, 8), lambda i: (i, 0)),
            pl.BlockSpec((8, 128), lambda i: (0, 0)),
        ],
        out_shape=[
            jax.ShapeDtypeStruct((n, 128), F32),
            jax.ShapeDtypeStruct((n, 8), F32),
            jax.ShapeDtypeStruct((8, 128), F32),
        ],
    )(x, W, A8)


def _tc_mid(xin, b, W, A8, n):
    return pl.pallas_call(
        _mid_body,
        grid=(n // BM,),
        in_specs=[
            pl.BlockSpec((BM, 128), lambda i: (i, 0)),
            pl.BlockSpec((1, 128), lambda i: (0, 0)),
            pl.BlockSpec((128, 128), lambda i: (0, 0)),
            pl.BlockSpec((128, 8), lambda i: (0, 0)),
        ],
        out_specs=[
            pl.BlockSpec((BM, 128), lambda i: (i, 0)),
            pl.BlockSpec((BM, 8), lambda i: (i, 0)),
            pl.BlockSpec((8, 128), lambda i: (0, 0)),
        ],
        out_shape=[
            jax.ShapeDtypeStruct((n, 128), F32),
            jax.ShapeDtypeStruct((n, 8), F32),
            jax.ShapeDtypeStruct((8, 128), F32),
        ],
    )(xin, b, W, A8)


def _tc_post(xin, b, Wc, bc, n, dout):
    return pl.pallas_call(
        _post_body,
        grid=(n // BM,),
        in_specs=[
            pl.BlockSpec((BM, 128), lambda i: (i, 0)),
            pl.BlockSpec((1, 128), lambda i: (0, 0)),
            pl.BlockSpec((128, dout), lambda i: (0, 0)),
            pl.BlockSpec((1, dout), lambda i: (0, 0)),
        ],
        out_specs=[
            pl.BlockSpec((BM, dout), lambda i: (i, 0)),
            pl.BlockSpec((BM, 128), lambda i: (i, 0)),
        ],
        out_shape=[
            jax.ShapeDtypeStruct((n, dout), F32),
            jax.ShapeDtypeStruct((n, 128), F32),
        ],
    )(xin, b, Wc, bc)


def _make_sc_scatter(npad, n_chunks):
    pt_pair = n_chunks // N_SUB
    pt0 = int(round(pt_pair * 0.613 / NRING)) * NRING
    pt1 = pt_pair - pt0
    slab = npad // N_SUB
    assert pt0 % NRING == 0 and pt1 % NRING == 0
    assert slab % 8 == 0 and CHUNK % 8 == 0

    mesh = plsc.VectorSubcoreMesh(core_axis_name="c", subcore_axis_name="s")

    @functools.partial(
        pl.kernel,
        mesh=mesh,
        compiler_params=_sc_compiler_params(),
        out_type=[
            jax.ShapeDtypeStruct((N_CORES, npad, 128), F32),
            jax.ShapeDtypeStruct((N_CORES * npad,), F32),
        ],
        scratch_types=[
            pltpu.VMEM((LANES,), F32),
            pltpu.VMEM((CHUNK,), jnp.int32),
            pltpu.VMEM((CHUNK,), jnp.int32),
            pltpu.VMEM((CHUNK,), jnp.int32),
            pltpu.VMEM((CHUNK,), jnp.int32),
            pltpu.VMEM((CHUNK,), jnp.int32),
            pltpu.VMEM((CHUNK,), jnp.int32),
            pltpu.VMEM((CHUNK,), F32),
            pltpu.VMEM((CHUNK,), F32),
            pltpu.VMEM((CHUNK,), F32),
            pltpu.VMEM((CHUNK,), F32),
            pltpu.VMEM((CHUNK,), F32),
            pltpu.VMEM((CHUNK,), F32),
            pltpu.VMEM((CHUNK, 128), F32),
            pltpu.VMEM((CHUNK, 128), F32),
            pltpu.VMEM((CHUNK, 128), F32),
            pltpu.VMEM_SHARED((npad, 128), F32),
            pltpu.VMEM_SHARED((npad,), F32),
            pltpu.SemaphoreType.DMA,
            pltpu.SemaphoreType.DMA,
            pltpu.SemaphoreType.DMA,
            pltpu.SemaphoreType.DMA,
            pltpu.SemaphoreType.DMA,
            pltpu.SemaphoreType.DMA,
        ],
    )
    def sc_scatter(h_hbm, as_hbm, ad_hbm, g_hbm, src_hbm, dst_hbm,
                   out_hbm, den_hbm,
                   g_t, srcC0, srcC1, srcC2, dstC0, dstC1, dstC2,
                   asg0, asg1, asg2, adg0, adg1, adg2,
                   rows0, rows1, rows2, sh_out, sh_den,
                   gC0, gC1, gC2, sC0, sC1, sC2):
        c = lax.axis_index("c")
        t = lax.axis_index("s")
        srcC = [srcC0, srcC1, srcC2]
        dstC = [dstC0, dstC1, dstC2]
        asg = [asg0, asg1, asg2]
        adg = [adg0, adg1, adg2]
        rows = [rows0, rows1, rows2]
        gC = [gC0, gC1, gC2]
        sC = [sC0, sC1, sC2]

        pltpu.sync_copy(g_hbm, g_t)
        gv = g_t[...]

        z16 = jnp.zeros((LANES,), F32)

        @pl.loop(0, CHUNK)
        def _(r):
            for f in range(128 // LANES):
                rows0[r, pl.ds(f * LANES, LANES)] = z16

        @pl.loop(0, CHUNK // LANES)
        def _(i):
            asg0[pl.ds(i * LANES, LANES)] = z16

        nfull = slab // CHUNK
        rem = slab - nfull * CHUNK
        for q in range(nfull):
            pltpu.sync_copy(rows0, sh_out.at[pl.ds(t * slab + q * CHUNK, CHUNK)])
            pltpu.sync_copy(asg0, sh_den.at[pl.ds(t * slab + q * CHUNK, CHUNK)])
        if rem:
            pltpu.sync_copy(rows0.at[pl.ds(0, rem)],
                            sh_out.at[pl.ds(t * slab + nfull * CHUNK, rem)])
            pltpu.sync_copy(asg0.at[pl.ds(0, rem)],
                            sh_den.at[pl.ds(t * slab + nfull * CHUNK, rem)])

        plsc.subcore_barrier()

        is0 = c == 0
        per_tile = jnp.where(is0, pt0, pt1)
        core_off = jnp.where(is0, 0, N_SUB * pt0)
        cbase = (core_off + t * per_tile) * CHUNK

        def idx_load(k, b):
            sl = pl.ds(cbase + k * CHUNK, CHUNK)
            pltpu.sync_copy(src_hbm.at[sl], srcC[b])
            pltpu.sync_copy(dst_hbm.at[sl], dstC[b])

        def gather_issue(b):
            pltpu.async_copy(h_hbm.at[srcC[b]], rows[b], gC[b])
            pltpu.async_copy(as_hbm.at[srcC[b]], asg[b], gC[b])
            pltpu.async_copy(ad_hbm.at[dstC[b]], adg[b], gC[b])

        def gather_wait(b):
            pltpu.make_async_copy(h_hbm.at[srcC[b]], rows[b], gC[b]).wait()
            pltpu.make_async_copy(as_hbm.at[srcC[b]], asg[b], gC[b]).wait()
            pltpu.make_async_copy(ad_hbm.at[dstC[b]], adg[b], gC[b]).wait()

        def scat_issue(b):
            pltpu.async_copy(rows[b], sh_out.at[dstC[b]], sC[b], add=True)
            pltpu.async_copy(asg[b], sh_den.at[dstC[b]], sC[b], add=True)

        def scat_drain(b):
            pltpu.make_async_copy(rows[b], sh_out.at[dstC[b]], sC[b]).wait()
            pltpu.make_async_copy(asg[b], sh_den.at[dstC[b]], sC[b]).wait()

        for p in range(2):
            idx_load(p, p)
            gather_issue(p)

        @pl.loop(0, per_tile // NRING)
        def _(m):
            for b in range(NRING):
                k = m * NRING + b
                b2 = (b + 2) % NRING
                gather_wait(b)

                @pl.loop(0, CHUNK // LANES)
                def _(i):
                    sl = pl.ds(i * LANES, LANES)
                    a = asg[b][sl] + adg[b][sl]
                    lr = jnp.maximum(a, 0.2 * a)
                    val = jnp.exp(lr - gv)
                    asg[b][sl] = val
                    for kk in range(LANES):
                        cs = val[kk]
                        r = i * LANES + kk
                        for f in range(128 // LANES):
                            fl = pl.ds(f * LANES, LANES)
                            rows[b][r, fl] = rows[b][r, fl] * cs

                scat_issue(b)

                @pl.when(k + 2 < per_tile)
                def _():
                    @pl.when(k >= 1)
                    def _():
                        scat_drain(b2)

                    idx_load(k + 2, b2)
                    gather_issue(b2)

        for b in range(NRING):
            scat_drain(b)

        plsc.subcore_barrier()
        pltpu.sync_copy(sh_out.at[pl.ds(t * slab, slab)],
                        out_hbm.at[c, pl.ds(t * slab, slab)])
        pltpu.sync_copy(sh_den.at[pl.ds(t * slab, slab)],
                        den_hbm.at[pl.ds(c * npad + t * slab, slab)])

    return sc_scatter


def _make_sc_finalize(npad):
    rpw = npad // NW
    assert rpw % LANES == 0

    mesh = plsc.VectorSubcoreMesh(core_axis_name="c", subcore_axis_name="s")

    @functools.partial(
        pl.kernel,
        mesh=mesh,
        compiler_params=_sc_compiler_params(),
        out_type=jax.ShapeDtypeStruct((npad, 128), F32),
        scratch_types=[
            pltpu.VMEM((rpw, 128), F32),
            pltpu.VMEM((rpw, 128), F32),
            pltpu.VMEM((rpw,), F32),
            pltpu.VMEM((rpw,), F32),
        ],
    )
    def sc_finalize(parts_hbm, den_hbm, out_hbm, pa, pb, da, db):
        c = lax.axis_index("c")
        t = lax.axis_index("s")
        w = c * N_SUB + t
        r0 = w * rpw
        pltpu.sync_copy(parts_hbm.at[0, pl.ds(r0, rpw)], pa)
        pltpu.sync_copy(parts_hbm.at[1, pl.ds(r0, rpw)], pb)
        pltpu.sync_copy(den_hbm.at[pl.ds(r0, rpw)], da)
        pltpu.sync_copy(den_hbm.at[pl.ds(npad + r0, rpw)], db)

        @pl.loop(0, rpw // LANES)
        def _(i):
            sl = pl.ds(i * LANES, LANES)
            inv = 1.0 / (da[sl] + db[sl] + 1e-16)
            for kk in range(LANES):
                cs = inv[kk]
                r = i * LANES + kk
                for f in range(128 // LANES):
                    fl = pl.ds(f * LANES, LANES)
                    pa[r, fl] = (pa[r, fl] + pb[r, fl]) * cs

        pltpu.sync_copy(pa, out_hbm.at[pl.ds(r0, rpw)])

    return sc_finalize


def kernel(fts, edge_index, W1, a_src1, a_dst1, b1, W2, a_src2, a_dst2, b2,
           Wc, bc):
    n, d_in = fts.shape
    e = edge_index.shape[1]
    dout = Wc.shape[1]

    npad = _cdiv(n + 1, 8 * NW) * (8 * NW)
    esl = e + n
    n_chunks = _cdiv(_cdiv(esl, CHUNK), NW * NRING) * (NW * NRING)
    epad = n_chunks * CHUNK

    loop = jnp.arange(n, dtype=jnp.int32)
    src = jnp.concatenate([edge_index[0].astype(jnp.int32), loop])
    dst = jnp.concatenate([edge_index[1].astype(jnp.int32), loop])
    pad_dst = n + (jnp.arange(epad - esl, dtype=jnp.int32) % (npad - n))
    srcp = jnp.pad(src, (0, epad - esl))
    dstp = jnp.concatenate([dst, pad_dst])

    def attn_mat(a_s, a_d):
        A8 = jnp.zeros((128, 8), F32)
        return A8.at[:, 0].set(a_s).at[:, 1].set(a_d)

    sc_scatter = _make_sc_scatter(npad, n_chunks)
    sc_finalize = _make_sc_finalize(npad)

    def edge_phase(h, aa, gm):
        asv = jnp.pad(aa[:, 0], (0, npad - n))
        adv = jnp.pad(aa[:, 1], (0, npad - n))
        s = gm[0, 0] + gm[1, 0]
        g16 = jnp.full((LANES,), jnp.maximum(s, 0.2 * s), F32)
        parts, dens = sc_scatter(h, asv, adv, g16, srcp, dstp)
        return sc_finalize(parts, dens)

    h1, aa1, gm1 = _tc_pre(fts, W1, attn_mat(a_src1, a_dst1), n)
    agg1 = edge_phase(h1, aa1, gm1)

    h2, aa2, gm2 = _tc_mid(agg1, b1.reshape(1, 128), W2,
                           attn_mat(a_src2, a_dst2), n)
    agg2 = edge_phase(h2, aa2, gm2)

    out, hf = _tc_post(agg2, b2.reshape(1, 128), Wc.astype(F32),
                       bc.reshape(1, dout), n, dout)
    return (out, hf)
